# Initial kernel scaffold; baseline (speedup 1.0000x reference)
import jax
import jax.numpy as jnp
from jax.experimental import pallas as pl


def kernel(x, edge_index, edge_attr, batch, W_init, b_init, Wl0, Wa0, Wl1, Wa1, Wl2, Wa2, Wn1, bn1, gn, betan, Wn2, bn2, Wg1, bg1, gg, betag, Wg2, bg2):
    raise NotImplementedError("write your pallas kernel here")


# speedup vs baseline: 7.5466x; 7.5466x over previous
import jax
import jax.numpy as jnp
from jax import lax
from jax.experimental import pallas as pl
from jax.experimental.pallas import tpu as pltpu
from jax.experimental.pallas import tpu_sc as plsc

N = 10000
NP = 10240
E = 320000
H = 128
NG = 64
NTILES = 32
FPT = H // NTILES
CH = 3200
NCH = E // CH
VPC = CH // 16
NEG_BIG = -3.0e38

_f32 = jnp.float32
_i32 = jnp.int32


def _sc_edge_body(nf_hbm, aa_hbm, mm_hbm, ep_hbm, u_hbm, s_hbm,
                  nf_v, aa_v, mm_v, u_v, s_v, eb0, eb1, sem0, sem1):
    c = lax.axis_index("c")
    sub = lax.axis_index("s")
    wid = sub * 2 + c

    pltpu.sync_copy(nf_hbm.at[pl.ds(wid * FPT * NP, FPT * NP)], nf_v)
    pltpu.sync_copy(aa_hbm.at[:], aa_v)
    pltpu.sync_copy(mm_hbm.at[:], mm_v)
    mtot = mm_v[...]

    def zb(i, _):
        u_v[pl.ds(i * 16, 16)] = jnp.zeros((16,), _f32)
        return 0
    lax.fori_loop(0, FPT * NP // 16, zb, 0)

    def zs(i, _):
        s_v[pl.ds(i * 16, 16)] = jnp.zeros((16,), _f32)
        return 0
    lax.fori_loop(0, NP // 16, zs, 0)

    def process(buf):
        def vbody(i, _):
            epk = buf[pl.ds(i * 16, 16)]
            src = lax.shift_right_logical(epk, 14)
            dst = lax.bitwise_and(epk, 16383)
            asrc = plsc.load_gather(aa_v, [src])
            adst = plsc.load_gather(aa_v, [dst + NP])
            w = jnp.exp((asrc - mtot) + adst)
            for r in range(FPT):
                v = plsc.load_gather(nf_v, [src + r * NP])
                plsc.addupdate_scatter(u_v, [dst + r * NP], v * w)
            plsc.addupdate_scatter(s_v, [dst], w)
            return 0
        lax.fori_loop(0, VPC, vbody, 0)

    pltpu.make_async_copy(ep_hbm.at[pl.ds(0, CH)], eb0, sem0).start()
    pltpu.make_async_copy(ep_hbm.at[pl.ds(CH, CH)], eb1, sem1).start()

    nouter = NCH // 2

    def outer(k, _):
        pltpu.make_async_copy(ep_hbm.at[pl.ds(0, CH)], eb0, sem0).wait()
        process(eb0)

        @pl.when(k < nouter - 1)
        def _():
            pltpu.make_async_copy(
                ep_hbm.at[pl.ds((2 * k + 2) * CH, CH)], eb0, sem0).start()

        pltpu.make_async_copy(ep_hbm.at[pl.ds(0, CH)], eb1, sem1).wait()
        process(eb1)

        @pl.when(k < nouter - 1)
        def _():
            pltpu.make_async_copy(
                ep_hbm.at[pl.ds((2 * k + 3) * CH, CH)], eb1, sem1).start()

        return 0

    lax.fori_loop(0, nouter, outer, 0)

    pltpu.sync_copy(u_v, u_hbm.at[pl.ds(wid * FPT * NP, FPT * NP)])

    @pl.when(wid == NTILES - 1)
    def _():
        pltpu.sync_copy(s_v, s_hbm.at[:])


_sc_edge = pl.kernel(
    _sc_edge_body,
    out_type=(jax.ShapeDtypeStruct((H * NP,), _f32),
              jax.ShapeDtypeStruct((NP,), _f32)),
    mesh=plsc.VectorSubcoreMesh(core_axis_name="c", subcore_axis_name="s"),
    scratch_types=[
        pltpu.VMEM((FPT * NP,), _f32),
        pltpu.VMEM((2 * NP,), _f32),
        pltpu.VMEM((16,), _f32),
        pltpu.VMEM((FPT * NP,), _f32),
        pltpu.VMEM((NP,), _f32),
        pltpu.VMEM((CH,), _i32),
        pltpu.VMEM((CH,), _i32),
        pltpu.SemaphoreType.DMA,
        pltpu.SemaphoreType.DMA,
    ],
    compiler_params=pltpu.CompilerParams(needs_layout_passes=False),
)


_bf16 = jnp.bfloat16


def _dotb(a, b, dims):
    return lax.dot_general(a.astype(_bf16), b.astype(_bf16), (dims, ((), ())),
                           preferred_element_type=_f32)

_NB = 2048
_GRID = NP // _NB


def _aa_and_max(nf, wa_ref, aa_ref, mm_ref, j):
    aab = _dotb(wa_ref[...], nf, ((1,), (0,)))
    aa_ref[...] = aab
    cur = jnp.max(aab, axis=1, keepdims=True)

    @pl.when(j == 0)
    def _():
        mm_ref[...] = cur

    @pl.when(j > 0)
    def _():
        mm_ref[...] = jnp.maximum(mm_ref[...], cur)


def _init_body(x_ref, wi_ref, bi_ref, wa_ref, nf_ref, aa_ref, mm_ref):
    j = pl.program_id(0)
    nf = _dotb(wi_ref[...], x_ref[...], ((1,), (1,))) + bi_ref[...]
    lane = j * _NB + lax.broadcasted_iota(_i32, (H, _NB), 1)
    nf = jnp.where(lane < N, nf, 0.0)
    nf_ref[...] = nf
    _aa_and_max(nf, wa_ref, aa_ref, mm_ref, j)


_init_tc = pl.pallas_call(
    _init_body,
    grid=(_GRID,),
    in_specs=[
        pl.BlockSpec((_NB, H), lambda j: (j, 0)),
        pl.BlockSpec((H, H), lambda j: (0, 0)),
        pl.BlockSpec((H, 1), lambda j: (0, 0)),
        pl.BlockSpec((2, H), lambda j: (0, 0)),
    ],
    out_specs=[
        pl.BlockSpec((H, _NB), lambda j: (0, j)),
        pl.BlockSpec((2, _NB), lambda j: (0, j)),
        pl.BlockSpec((2, 1), lambda j: (0, 0)),
    ],
    out_shape=[
        jax.ShapeDtypeStruct((H, NP), _f32),
        jax.ShapeDtypeStruct((2, NP), _f32),
        jax.ShapeDtypeStruct((2, 1), _f32),
    ],
)


def _layer_body(nf_ref, u_ref, s_ref, wll_ref, wlr_ref, wa_ref,
                nfo_ref, aao_ref, mm_ref):
    j = pl.program_id(0)
    agg = u_ref[...] * (1.0 / (s_ref[...] + 1e-30))
    h = (_dotb(wll_ref[...], nf_ref[...], ((1,), (0,)))
         + _dotb(wlr_ref[...], agg, ((1,), (0,))))
    nf2 = jnp.maximum(h, 0.0)
    nfo_ref[...] = nf2
    _aa_and_max(nf2, wa_ref, aao_ref, mm_ref, j)


_layer_tc = pl.pallas_call(
    _layer_body,
    grid=(_GRID,),
    in_specs=[
        pl.BlockSpec((H, _NB), lambda j: (0, j)),
        pl.BlockSpec((H, _NB), lambda j: (0, j)),
        pl.BlockSpec((1, _NB), lambda j: (0, j)),
        pl.BlockSpec((H, H), lambda j: (0, 0)),
        pl.BlockSpec((H, H), lambda j: (0, 0)),
        pl.BlockSpec((2, H), lambda j: (0, 0)),
    ],
    out_specs=[
        pl.BlockSpec((H, _NB), lambda j: (0, j)),
        pl.BlockSpec((2, _NB), lambda j: (0, j)),
        pl.BlockSpec((2, 1), lambda j: (0, 0)),
    ],
    out_shape=[
        jax.ShapeDtypeStruct((H, NP), _f32),
        jax.ShapeDtypeStruct((2, NP), _f32),
        jax.ShapeDtypeStruct((2, 1), _f32),
    ],
)


def _final_body(nf_ref, b_ref, wn1_ref, bn1_ref, gn_ref, betan_ref,
                wn2_ref, bn2_ref, wg1_ref, bg1_ref, gg_ref, betag_ref,
                wg2_ref, bg2_ref, nfo_ref, gfo_ref):
    nf = nf_ref[...]
    bt = b_ref[...]
    gid = lax.broadcasted_iota(_i32, (NG, NP), 0)
    BT = (gid == bt).astype(_f32)

    q = jnp.sum(nf * nf, axis=0, keepdims=True)
    l = q
    a = q
    for _ in range(3):
        masked = jnp.where(BT > 0.0, l, NEG_BIG)
        m = jnp.max(masked, axis=1, keepdims=True)
        m = jnp.where(m > NEG_BIG * 0.5, m, 0.0)
        mn = lax.dot_general(m, BT, (((0,), (0,)), ((), ())),
                             preferred_element_type=_f32, precision=lax.Precision.HIGHEST)
        e = jnp.exp(l - mn)
        sseg = lax.dot_general(BT, e, (((1,), (1,)), ((), ())),
                               preferred_element_type=_f32, precision=lax.Precision.HIGHEST)
        sn = lax.dot_general(sseg, BT, (((0,), (0,)), ((), ())),
                             preferred_element_type=_f32, precision=lax.Precision.HIGHEST)
        a = e / (sn + 1e-16)
        l = a * q

    gfT = lax.dot_general(nf * a, BT, (((1,), (1,)), ((), ())),
                          preferred_element_type=_f32, precision=lax.Precision.HIGHEST)

    vmask = (lax.broadcasted_iota(_i32, (H, NP), 1) < N).astype(_f32)

    z = _dotb(wn1_ref[...], nf, ((1,), (0,))) + bn1_ref[...]
    mu = jnp.sum(z * vmask, axis=1, keepdims=True) * (1.0 / N)
    zc = z - mu
    var = jnp.sum(zc * zc * vmask, axis=1, keepdims=True) * (1.0 / N)
    zh = gn_ref[...] * zc * lax.rsqrt(var + 1e-5) + betan_ref[...]
    r = jnp.maximum(zh, 0.0)
    nfo_ref[...] = _dotb(r, wn2_ref[...], ((0,), (1,))) + bn2_ref[...]

    z2 = _dotb(wg1_ref[...], gfT, ((1,), (0,))) + bg1_ref[...]
    mu2 = jnp.sum(z2, axis=1, keepdims=True) * (1.0 / NG)
    zc2 = z2 - mu2
    var2 = jnp.sum(zc2 * zc2, axis=1, keepdims=True) * (1.0 / NG)
    zh2 = gg_ref[...] * zc2 * lax.rsqrt(var2 + 1e-5) + betag_ref[...]
    r2 = jnp.maximum(zh2, 0.0)
    gfo_ref[...] = _dotb(r2, wg2_ref[...], ((0,), (1,))) + bg2_ref[...]


_final_tc = pl.pallas_call(
    _final_body,
    out_shape=[
        jax.ShapeDtypeStruct((NP, H), _f32),
        jax.ShapeDtypeStruct((NG, NG), _f32),
    ],
)


def kernel(x, edge_index, edge_attr, batch, W_init, b_init, Wl0, Wa0, Wl1,
           Wa1, Wl2, Wa2, Wn1, bn1, gn, betan, Wn2, bn2, Wg1, bg1, gg, betag,
           Wg2, bg2):
    src = edge_index[0].astype(_i32)
    dst = edge_index[1].astype(_i32)
    ep = src * 16384 + dst

    xp = jnp.pad(x, ((0, NP - N), (0, 0)))
    bp = jnp.pad(batch.astype(_i32), (0, NP - N),
                 constant_values=NG).reshape(1, NP)

    nf, aa, mm = _init_tc(xp, W_init, b_init.reshape(H, 1), Wa0.reshape(2, H))

    was = (Wa0.reshape(2, H), Wa1.reshape(2, H), Wa2.reshape(2, H))
    wls = (Wl0, Wl1, Wl2)
    for i in range(3):
        mvec = jnp.broadcast_to(mm[0, 0] + mm[1, 0], (16,))
        u, s = _sc_edge(nf.reshape(H * NP), aa.reshape(2 * NP), mvec, ep)
        wa_next = was[i + 1] if i < 2 else was[0]
        nf, aa, mm = _layer_tc(nf, u.reshape(H, NP), s.reshape(1, NP),
                               wls[i][:, :H], wls[i][:, H:], wa_next)

    nf_o, gf_o = _final_tc(
        nf, bp,
        Wn1, bn1.reshape(H, 1), gn.reshape(H, 1), betan.reshape(H, 1),
        Wn2, bn2.reshape(1, H),
        Wg1, bg1.reshape(H, 1), gg.reshape(H, 1), betag.reshape(H, 1),
        Wg2, bg2.reshape(1, NG))
    return (nf_o[:N], edge_attr, gf_o)
